# baseline (speedup 1.0000x reference)
import functools
import math

import jax
import jax.numpy as jnp
from jax import lax
from jax.experimental import pallas as pl
from jax.experimental.pallas import tpu as pltpu
from jax.experimental.pallas import tpu_sc as plsc

_V = 100000
_B = 1024
_S = 0.1 / (_V - 2)
_C = 0.9
_ENT = _C * math.log(_C) + (_V - 2) * _S * math.log(_S)

_BV = 2048
_NK = (_V + _BV - 1) // _BV
_CPB = _BV // 128

_NC = 2
_NS = 16
_NW = _NC * _NS
_BPW = _B // _NW


def _sc_gather_body(t_hbm, xflat_hbm, g_hbm, t_v, idx_v, g_v, sem):
    wid = lax.axis_index("s") * _NC + lax.axis_index("c")
    base = wid * _BPW
    pltpu.sync_copy(t_hbm.at[pl.ds(base, _BPW)], t_v)
    for c in range(_BPW // 16):
        tv = t_v[pl.ds(c * 16, 16)]
        rows = base + c * 16 + lax.iota(jnp.int32, 16)
        idx_v[pl.ds(c * 16, 16)] = rows * _V + tv
    pltpu.async_copy(xflat_hbm.at[idx_v], g_v, sem).wait()
    pltpu.sync_copy(g_v, g_hbm.at[pl.ds(base, _BPW)])


@functools.cache
def _sc_gather():
    return pl.kernel(
        _sc_gather_body,
        out_type=jax.ShapeDtypeStruct((_B,), jnp.float32),
        mesh=plsc.VectorSubcoreMesh(
            core_axis_name="c", subcore_axis_name="s", num_cores=_NC,
            num_subcores=_NS),
        scratch_types=[
            pltpu.VMEM((_BPW,), jnp.int32),
            pltpu.VMEM((_BPW,), jnp.int32),
            pltpu.VMEM((_BPW,), jnp.float32),
            pltpu.SemaphoreType.DMA,
        ],
    )


def _tc_body(t_ref, g_ref, x_ref, o_ref, acc_ref):
    k = pl.program_id(0)

    @pl.when(k == 0)
    def _first():
        lane = lax.broadcasted_iota(jnp.int32, (_B, 128), 1)
        b = jnp.where(lane == 0, 0.0, x_ref[:, 0:128])
        for c in range(1, _CPB):
            b = b + x_ref[:, c * 128:(c + 1) * 128]
        acc_ref[...] = b

    @pl.when((k > 0) & (k < _NK - 1))
    def _mid():
        b = x_ref[:, 0:128]
        for c in range(1, _CPB):
            b = b + x_ref[:, c * 128:(c + 1) * 128]
        acc_ref[...] += b

    @pl.when(k == _NK - 1)
    def _last():
        nvalid = _V - (_NK - 1) * _BV
        full = nvalid // 128
        rem = nvalid - full * 128
        b = x_ref[:, 0:128]
        for c in range(1, full):
            b = b + x_ref[:, c * 128:(c + 1) * 128]
        if rem:
            lane = lax.broadcasted_iota(jnp.int32, (_B, 128), 1)
            b = b + jnp.where(lane < rem,
                              x_ref[:, full * 128:(full + 1) * 128], 0.0)
        acc = acc_ref[...] + b
        rows = jnp.sum(acc, axis=1, keepdims=True)
        t = t_ref[...]
        g = g_ref[...]
        per = jnp.where(t != 0,
                        _ENT - (_C - _S) * g - _S * rows,
                        0.0).astype(jnp.float32)
        o_ref[0, 0] = jnp.sum(per)


def _tc_reduce(t2, g2, x):
    out = pl.pallas_call(
        _tc_body,
        grid=(_NK,),
        in_specs=[
            pl.BlockSpec((_B, 1), lambda k: (0, 0)),
            pl.BlockSpec((_B, 1), lambda k: (0, 0)),
            pl.BlockSpec((_B, _BV), lambda k: (0, k)),
        ],
        out_specs=pl.BlockSpec(memory_space=pltpu.SMEM),
        out_shape=jax.ShapeDtypeStruct((1, 1), jnp.float32),
        scratch_shapes=[pltpu.VMEM((_B, 128), jnp.float32)],
        compiler_params=pltpu.CompilerParams(
            dimension_semantics=("arbitrary",),
        ),
    )(t2, g2, x)
    return out[0, 0]


@jax.jit
def kernel(x, target):
    t32 = target.astype(jnp.int32)
    g = _sc_gather()(t32, x.reshape(-1))
    return _tc_reduce(t32.reshape(_B, 1), g.reshape(_B, 1), x)


# speedup vs baseline: 1.9348x; 1.9348x over previous
import functools
import math

import jax
import jax.numpy as jnp
from jax import lax
from jax.experimental import pallas as pl
from jax.experimental.pallas import tpu as pltpu

_V = 100000
_B = 1024
_S = 0.1 / (_V - 2)
_C = 0.9
_ENT = _C * math.log(_C) + (_V - 2) * _S * math.log(_S)

_BR = 8
_NR = _B // _BR
_NFULL = _V // 128
_REM = _V - _NFULL * 128


def _body(t_ref, x_ref, o_ref):
    i = pl.program_id(0)
    t = t_ref[...]
    mask = t != 0
    lane = lax.broadcasted_iota(jnp.int32, (_BR, 128), 1)
    ch0 = x_ref[:, 0:128]
    acc = ch0
    gacc = jnp.where(lane == t, ch0, 0.0)
    for c in range(1, _NFULL):
        ch = x_ref[:, c * 128:(c + 1) * 128]
        acc = acc + ch
        gacc = gacc + jnp.where(lane == t - c * 128, ch, 0.0)
    rs = jnp.sum(acc, axis=1, keepdims=True)
    gv = jnp.sum(gacc, axis=1, keepdims=True)
    if _REM:
        tch = x_ref[:, _NFULL * 128:_V]
        lane_t = lax.broadcasted_iota(jnp.int32, (_BR, _REM), 1)
        rs = rs + jnp.sum(tch, axis=1, keepdims=True)
        gv = gv + jnp.sum(
            jnp.where(lane_t == t - _NFULL * 128, tch, 0.0),
            axis=1, keepdims=True)
    x0 = x_ref[:, 0:1]
    per = jnp.where(mask, _ENT - (_C - _S) * gv - _S * (rs - x0), 0.0)
    partial = jnp.sum(per.astype(jnp.float32))

    @pl.when(i == 0)
    def _init():
        o_ref[0, 0] = partial

    @pl.when(i > 0)
    def _acc():
        o_ref[0, 0] += partial


def _tc_all(t2, x):
    out = pl.pallas_call(
        _body,
        grid=(_NR,),
        in_specs=[
            pl.BlockSpec((_BR, 1), lambda i: (i, 0)),
            pl.BlockSpec((_BR, _V), lambda i: (i, 0)),
        ],
        out_specs=pl.BlockSpec(memory_space=pltpu.SMEM),
        out_shape=jax.ShapeDtypeStruct((1, 1), jnp.float32),
        compiler_params=pltpu.CompilerParams(
            dimension_semantics=("arbitrary",),
        ),
    )(t2, x)
    return out[0, 0]


@jax.jit
def kernel(x, target):
    return _tc_all(target.astype(jnp.int32).reshape(_B, 1), x)


# speedup vs baseline: 2.1382x; 1.1051x over previous
import functools
import math

import jax
import jax.numpy as jnp
from jax import lax
from jax.experimental import pallas as pl
from jax.experimental.pallas import tpu as pltpu

_V = 100000
_B = 1024
_S = 0.1 / (_V - 2)
_C = 0.9
_ENT = _C * math.log(_C) + (_V - 2) * _S * math.log(_S)

_BR = 32
_NR = _B // _BR
_NFULL = _V // 128
_REM = _V - _NFULL * 128


def _body(t_ref, x_ref, o_ref):
    i = pl.program_id(0)
    t = t_ref[...]
    mask = t != 0
    lane = lax.broadcasted_iota(jnp.int32, (_BR, 128), 1)
    ch0 = x_ref[:, 0:128]
    acc = ch0
    gacc = jnp.where(lane == t, ch0, 0.0)
    for c in range(1, _NFULL):
        ch = x_ref[:, c * 128:(c + 1) * 128]
        acc = acc + ch
        gacc = gacc + jnp.where(lane == t - c * 128, ch, 0.0)
    rs = jnp.sum(acc, axis=1, keepdims=True)
    gv = jnp.sum(gacc, axis=1, keepdims=True)
    if _REM:
        tch = x_ref[:, _NFULL * 128:_V]
        lane_t = lax.broadcasted_iota(jnp.int32, (_BR, _REM), 1)
        rs = rs + jnp.sum(tch, axis=1, keepdims=True)
        gv = gv + jnp.sum(
            jnp.where(lane_t == t - _NFULL * 128, tch, 0.0),
            axis=1, keepdims=True)
    x0 = x_ref[:, 0:1]
    per = jnp.where(mask, _ENT - (_C - _S) * gv - _S * (rs - x0), 0.0)
    partial = jnp.sum(per.astype(jnp.float32))

    @pl.when(i == 0)
    def _init():
        o_ref[0, 0] = partial

    @pl.when(i > 0)
    def _acc():
        o_ref[0, 0] += partial


def _tc_all(t2, x):
    out = pl.pallas_call(
        _body,
        grid=(_NR,),
        in_specs=[
            pl.BlockSpec((_BR, 1), lambda i: (i, 0)),
            pl.BlockSpec((_BR, _V), lambda i: (i, 0)),
        ],
        out_specs=pl.BlockSpec(memory_space=pltpu.SMEM),
        out_shape=jax.ShapeDtypeStruct((1, 1), jnp.float32),
        compiler_params=pltpu.CompilerParams(
            dimension_semantics=("arbitrary",),
        ),
    )(t2, x)
    return out[0, 0]


@jax.jit
def kernel(x, target):
    return _tc_all(target.astype(jnp.int32).reshape(_B, 1), x)
